# baseline (speedup 1.0000x reference)
import functools

import jax
import jax.numpy as jnp
from jax import lax
from jax.experimental import pallas as pl
from jax.experimental.pallas import tpu as pltpu
from jax.experimental.pallas import tpu_sc as plsc

NC = 2
NS = 16
NW = NC * NS
CH = 80
NUM_GRAPHS = 128

_HI = jax.lax.Precision.HIGHEST


def _make_deg_kernel(E, N):
    epw = E // NW
    mesh = plsc.VectorSubcoreMesh(core_axis_name="c", subcore_axis_name="s")

    @functools.partial(
        pl.kernel,
        out_type=jax.ShapeDtypeStruct((NW * N,), jnp.float32),
        mesh=mesh,
        compiler_params=pltpu.CompilerParams(needs_layout_passes=False),
        scratch_types=[
            pltpu.VMEM((epw,), jnp.int32),
            pltpu.VMEM((N,), jnp.float32),
        ],
    )
    def deg_kernel(dst_hbm, out_hbm, dbuf, hist):
        c = lax.axis_index("c")
        s = lax.axis_index("s")
        wid = c * NS + s
        pltpu.sync_copy(dst_hbm.at[pl.ds(wid * epw, epw)], dbuf)

        zeros16 = jnp.zeros((16,), jnp.float32)

        def zbody(i, carry):
            hist[pl.ds(i * 16, 16)] = zeros16
            return carry

        lax.fori_loop(0, N // 16, zbody, 0)

        ones16 = jnp.full((16,), 1.0, jnp.float32)

        def body(i, carry):
            idx = dbuf[pl.ds(i * 16, 16)]
            plsc.addupdate_scatter(hist, [idx], ones16)
            return carry

        lax.fori_loop(0, epw // 16, body, 0)
        pltpu.sync_copy(hist, out_hbm.at[pl.ds(wid * N, N)])

    return deg_kernel


def _make_edge_kernel(E, N, D):
    epw = E // NW
    ch = 125
    nch = epw // ch
    nhalf = nch // 2
    assert nch * ch == epw and nhalf % 2 == 0 and (wid_align := nch % 8) == 0
    rpw = (N // NS) // 8 * 8
    tail = N - NS * rpw
    mesh = plsc.VectorSubcoreMesh(core_axis_name="c", subcore_axis_name="s")

    @functools.partial(
        pl.kernel,
        out_type=jax.ShapeDtypeStruct((NC, N, D), jnp.float32),
        mesh=mesh,
        scratch_types=[
            pltpu.VMEM((nhalf, ch), jnp.int32),
            pltpu.VMEM((nhalf, ch), jnp.int32),
            pltpu.VMEM((ch, D), jnp.float32),
            pltpu.VMEM((ch, D), jnp.float32),
            pltpu.VMEM_SHARED((N, D), jnp.float32),
            pltpu.SemaphoreType.DMA,
            pltpu.SemaphoreType.DMA,
            pltpu.SemaphoreType.DMA,
            pltpu.SemaphoreType.DMA,
        ],
    )
    def edge_kernel(t_hbm, src_hbm, dst_hbm, z_hbm, out_hbm,
                    sidx, didx, rows0, rows1, acc, sem0, sem1, sem2, sem3):
        c = lax.axis_index("c")
        s = lax.axis_index("s")
        wid = c * NS + s
        row0 = s * rpw
        pltpu.sync_copy(z_hbm.at[pl.ds(row0, rpw)], acc.at[pl.ds(row0, rpw)])
        if tail:
            @pl.when(s == NS - 1)
            def _():
                pltpu.sync_copy(z_hbm.at[pl.ds(NS * rpw, tail)],
                                acc.at[pl.ds(NS * rpw, tail)])
        plsc.subcore_barrier()

        for half in range(2):
            cb = wid * nch + half * nhalf
            pltpu.sync_copy(src_hbm.at[pl.ds(cb, nhalf)], sidx)
            pltpu.sync_copy(dst_hbm.at[pl.ds(cb, nhalf)], didx)

            pltpu.async_copy(t_hbm.at[sidx.at[0]], rows0, sem0)
            pltpu.async_copy(t_hbm.at[sidx.at[1]], rows1, sem1)

            def body(jj, carry):
                j = jj * 2
                pltpu.make_async_copy(t_hbm.at[sidx.at[j]], rows0,
                                      sem0).wait()
                pltpu.async_copy(rows0, acc.at[didx.at[j]], sem2, add=True)
                pltpu.make_async_copy(t_hbm.at[sidx.at[j + 1]], rows1,
                                      sem1).wait()
                pltpu.async_copy(rows1, acc.at[didx.at[j + 1]], sem3,
                                 add=True)
                pltpu.make_async_copy(rows0, acc.at[didx.at[j]], sem2).wait()

                @pl.when(jj + 1 < nhalf // 2)
                def _():
                    pltpu.async_copy(t_hbm.at[sidx.at[j + 2]], rows0, sem0)

                pltpu.make_async_copy(rows1, acc.at[didx.at[j + 1]],
                                      sem3).wait()

                @pl.when(jj + 1 < nhalf // 2)
                def _():
                    pltpu.async_copy(t_hbm.at[sidx.at[j + 3]], rows1, sem1)

                return carry

            lax.fori_loop(0, nhalf // 2, body, 0)
        plsc.subcore_barrier()
        pltpu.sync_copy(acc.at[pl.ds(row0, rpw)],
                        out_hbm.at[c, pl.ds(row0, rpw)])
        if tail:
            @pl.when(s == NS - 1)
            def _():
                pltpu.sync_copy(acc.at[pl.ds(NS * rpw, tail)],
                                out_hbm.at[c, pl.ds(NS * rpw, tail)])

    return edge_kernel


BS = 2000

def _row_spec():
    return pl.BlockSpec((BS, 128), lambda i: (i, 0))


def _full_spec(shape):
    nd = len(shape)
    return pl.BlockSpec(shape, lambda i: (0,) * nd)


def _pre_body(x_ref, pw_ref, pb_ref, w0_ref, degp_ref,
              xw0_ref, dinv_ref, t0_ref):
    h = jnp.dot(x_ref[...], pw_ref[...], precision=_HI) + pb_ref[...]
    xw0 = jnp.dot(h, w0_ref[...], precision=_HI)
    xw0_ref[...] = xw0
    deg = jnp.sum(degp_ref[...], axis=0) + 1.0
    dinv = lax.rsqrt(deg)[:, None]
    dinv_ref[...] = dinv
    t0_ref[...] = dinv * xw0


def _layer_norm_relu(pre, g, beta):
    mu = jnp.mean(pre, axis=-1, keepdims=True)
    var = jnp.mean(jnp.square(pre - mu), axis=-1, keepdims=True)
    o = (pre - mu) * lax.rsqrt(var + 1e-5) * g + beta
    return jnp.maximum(o, 0.0)


def _make_cell_body(has_hsum_in, want_hsum_out):
    def body(*refs):
        (agg_ref, xw_ref, dinv_ref, b_ref, g_ref, beta_ref) = refs[:6]
        pos = 6
        if has_hsum_in:
            hin_ref = refs[pos]
            pos += 1
        wn_ref = refs[pos]
        pos += 1
        t_ref, xwn_ref = refs[pos], refs[pos + 1]
        pos += 2
        if want_hsum_out:
            hout_ref = refs[pos]

        dinv = dinv_ref[...]
        agg = agg_ref[...]
        a = agg[0] + agg[1]
        pre = dinv * a + (dinv * dinv) * xw_ref[...] + b_ref[...]
        o = _layer_norm_relu(pre, g_ref[...], beta_ref[...])
        if has_hsum_in:
            hsum = hin_ref[...] + o
        else:
            hsum = o
        if want_hsum_out:
            hout_ref[...] = hsum
        xwn = jnp.dot(hsum, wn_ref[...], precision=_HI)
        xwn_ref[...] = xwn
        t_ref[...] = dinv * xwn

    return body


def _final_body(agg_ref, xw_ref, dinv_ref, b_ref, g_ref, beta_ref,
                batch_ref, pw_ref, pb_ref, out_ref, pool_acc):
    i = pl.program_id(0)
    dinv = dinv_ref[...]
    agg = agg_ref[...]
    a = agg[0] + agg[1]
    pre = dinv * a + (dinv * dinv) * xw_ref[...] + b_ref[...]
    o = _layer_norm_relu(pre, g_ref[...], beta_ref[...])
    gid = lax.broadcasted_iota(jnp.int32, (1, NUM_GRAPHS), 1)
    onehot = (batch_ref[...] == gid).astype(jnp.float32)
    pooled = lax.dot_general(onehot, o, (((0,), (0,)), ((), ())),
                             precision=_HI)

    @pl.when(i == 0)
    def _():
        pool_acc[...] = jnp.zeros_like(pool_acc)

    pool_acc[...] += pooled

    @pl.when(i == pl.num_programs(0) - 1)
    def _():
        out_ref[...] = (jnp.dot(pool_acc[...], pw_ref[...], precision=_HI)
                        + pb_ref[...])


def _tc(body, grid, in_specs, out_specs, out_shape, *args, scratch_shapes=()):
    return pl.pallas_call(
        body, grid=grid, in_specs=in_specs, out_specs=out_specs,
        out_shape=out_shape, scratch_shapes=scratch_shapes,
        compiler_params=pltpu.CompilerParams(
            vmem_limit_bytes=60 * 1024 * 1024),
    )(*args)


def kernel(x, edge_index, batch, params):
    N, D = x.shape
    E = edge_index.shape[1]
    ch = 125
    src = edge_index[0].reshape(E // ch, ch)
    dst_flat = edge_index[1]
    dst = dst_flat.reshape(E // ch, ch)
    zeros_nd = jnp.zeros((N, D), jnp.float32)
    batch2 = batch.reshape(N, 1)
    cells = params["cells"]
    ncells = len(cells)

    deg_kernel = _make_deg_kernel(E, N)
    edge_kernel = _make_edge_kernel(E, N, D)

    deg_p = deg_kernel(dst_flat).reshape(NW, N)
    grid = (N // BS,)
    row = _row_spec
    dinv_spec = pl.BlockSpec((BS, 1), lambda i: (i, 0))
    w_spec = _full_spec((D, D))
    v_spec = _full_spec((D,))
    agg_spec = pl.BlockSpec((NC, BS, D), lambda i: (0, i, 0))
    nd_sds = jax.ShapeDtypeStruct((N, D), jnp.float32)

    xw, dinv, t = pl.pallas_call(
        _pre_body,
        out_shape=(nd_sds, jax.ShapeDtypeStruct((N, 1), jnp.float32),
                   nd_sds),
        compiler_params=pltpu.CompilerParams(
            vmem_limit_bytes=60 * 1024 * 1024),
    )(x, params["pre_W"], params["pre_b"], cells[0]["W"], deg_p)

    hsum = None
    for i in range(ncells):
        agg = edge_kernel(t, src, dst, zeros_nd)
        c = cells[i]
        if i < ncells - 1:
            has_hin = i > 0
            want_hout = i < ncells - 2
            ins = [agg, xw, dinv, c["b"], c["g"], c["beta"]]
            specs = [agg_spec, row(), dinv_spec, v_spec, v_spec, v_spec]
            if has_hin:
                ins.append(hsum)
                specs.append(row())
            ins.append(cells[i + 1]["W"])
            specs.append(w_spec)
            outs = [nd_sds, nd_sds]
            out_specs = [row(), row()]
            if want_hout:
                outs.append(nd_sds)
                out_specs.append(row())
            res = _tc(_make_cell_body(has_hin, want_hout), grid,
                      specs, tuple(out_specs), tuple(outs), *ins)
            if want_hout:
                t, xw, hsum = res
            else:
                t, xw = res
        else:
            n_out = params["post_W"].shape[1]
            out = _tc(
                _final_body, grid,
                [agg_spec, row(), dinv_spec, v_spec, v_spec, v_spec,
                 pl.BlockSpec((BS, 1), lambda i: (i, 0)),
                 _full_spec((D, n_out)), _full_spec((n_out,))],
                pl.BlockSpec((NUM_GRAPHS, n_out), lambda i: (0, 0)),
                jax.ShapeDtypeStruct((NUM_GRAPHS, n_out), jnp.float32),
                agg, xw, dinv, c["b"], c["g"], c["beta"], batch2,
                params["post_W"], params["post_b"],
                scratch_shapes=[pltpu.VMEM((NUM_GRAPHS, D), jnp.float32)])
    return out


# speedup vs baseline: 1.2467x; 1.2467x over previous
import functools

import jax
import jax.numpy as jnp
from jax import lax
from jax.experimental import pallas as pl
from jax.experimental.pallas import tpu as pltpu
from jax.experimental.pallas import tpu_sc as plsc

NC = 2
NS = 16
NW = NC * NS
CH = 80
NUM_GRAPHS = 128

_HI = jax.lax.Precision.HIGHEST


def _make_deg_kernel(E, N):
    epw = E // NW
    mesh = plsc.VectorSubcoreMesh(core_axis_name="c", subcore_axis_name="s")

    @functools.partial(
        pl.kernel,
        out_type=jax.ShapeDtypeStruct((NW * N,), jnp.float32),
        mesh=mesh,
        compiler_params=pltpu.CompilerParams(needs_layout_passes=False),
        scratch_types=[
            pltpu.VMEM((epw,), jnp.int32),
            pltpu.VMEM((N,), jnp.float32),
        ],
    )
    def deg_kernel(dst_hbm, out_hbm, dbuf, hist):
        c = lax.axis_index("c")
        s = lax.axis_index("s")
        wid = c * NS + s
        pltpu.sync_copy(dst_hbm.at[pl.ds(wid * epw, epw)], dbuf)

        zeros16 = jnp.zeros((16,), jnp.float32)

        def zbody(i, carry):
            hist[pl.ds(i * 16, 16)] = zeros16
            return carry

        lax.fori_loop(0, N // 16, zbody, 0)

        ones16 = jnp.full((16,), 1.0, jnp.float32)

        def body(i, carry):
            idx = dbuf[pl.ds(i * 16, 16)]
            plsc.addupdate_scatter(hist, [idx], ones16)
            return carry

        lax.fori_loop(0, epw // 16, body, 0)
        pltpu.sync_copy(hist, out_hbm.at[pl.ds(wid * N, N)])

    return deg_kernel


def _make_edge_kernel(E, N, D):
    epw = E // NW
    ch = 125
    nch = epw // ch
    nhalf = nch // 2
    assert nch * ch == epw and nhalf % 2 == 0 and (wid_align := nch % 8) == 0
    rpw = (N // NS) // 8 * 8
    tail = N - NS * rpw
    mesh = plsc.VectorSubcoreMesh(core_axis_name="c", subcore_axis_name="s")

    @functools.partial(
        pl.kernel,
        out_type=jax.ShapeDtypeStruct((NC, N, D), jnp.float32),
        mesh=mesh,
        scratch_types=[
            pltpu.VMEM((nhalf, ch), jnp.int32),
            pltpu.VMEM((nhalf, ch), jnp.int32),
            pltpu.VMEM((ch, D), jnp.float32),
            pltpu.VMEM((ch, D), jnp.float32),
            pltpu.VMEM_SHARED((N, D), jnp.float32),
            pltpu.SemaphoreType.DMA,
            pltpu.SemaphoreType.DMA,
            pltpu.SemaphoreType.DMA,
            pltpu.SemaphoreType.DMA,
        ],
    )
    def edge_kernel(t_hbm, src_hbm, dst_hbm, z_hbm, out_hbm,
                    sidx, didx, rows0, rows1, acc, sem0, sem1, sem2, sem3):
        c = lax.axis_index("c")
        s = lax.axis_index("s")
        wid = c * NS + s
        row0 = s * rpw
        pltpu.sync_copy(z_hbm.at[pl.ds(row0, rpw)], acc.at[pl.ds(row0, rpw)])
        if tail:
            @pl.when(s == NS - 1)
            def _():
                pltpu.sync_copy(z_hbm.at[pl.ds(NS * rpw, tail)],
                                acc.at[pl.ds(NS * rpw, tail)])
        plsc.subcore_barrier()

        for half in range(2):
            cb = wid * nch + half * nhalf
            pltpu.sync_copy(src_hbm.at[pl.ds(cb, nhalf)], sidx)
            pltpu.sync_copy(dst_hbm.at[pl.ds(cb, nhalf)], didx)

            pltpu.async_copy(t_hbm.at[sidx.at[0]], rows0, sem0)

            def body(jj, carry):
                j = jj * 2
                pltpu.async_copy(t_hbm.at[sidx.at[j + 1]], rows1, sem1)
                pltpu.make_async_copy(t_hbm.at[sidx.at[j]], rows0,
                                      sem0).wait()
                pltpu.sync_copy(rows0, acc.at[didx.at[j]], add=True)

                @pl.when(jj + 1 < nhalf // 2)
                def _():
                    pltpu.async_copy(t_hbm.at[sidx.at[j + 2]], rows0, sem0)

                pltpu.make_async_copy(t_hbm.at[sidx.at[j + 1]], rows1,
                                      sem1).wait()
                pltpu.sync_copy(rows1, acc.at[didx.at[j + 1]], add=True)
                return carry

            lax.fori_loop(0, nhalf // 2, body, 0)
        plsc.subcore_barrier()
        pltpu.sync_copy(acc.at[pl.ds(row0, rpw)],
                        out_hbm.at[c, pl.ds(row0, rpw)])
        if tail:
            @pl.when(s == NS - 1)
            def _():
                pltpu.sync_copy(acc.at[pl.ds(NS * rpw, tail)],
                                out_hbm.at[c, pl.ds(NS * rpw, tail)])

    return edge_kernel


BS = 2000

def _row_spec():
    return pl.BlockSpec((BS, 128), lambda i: (i, 0))


def _full_spec(shape):
    nd = len(shape)
    return pl.BlockSpec(shape, lambda i: (0,) * nd)


def _pre_body(x_ref, pw_ref, pb_ref, w0_ref, degp_ref,
              xw0_ref, dinv_ref, t0_ref):
    h = jnp.dot(x_ref[...], pw_ref[...], precision=_HI) + pb_ref[...]
    xw0 = jnp.dot(h, w0_ref[...], precision=_HI)
    xw0_ref[...] = xw0
    deg = jnp.sum(degp_ref[...], axis=0) + 1.0
    dinv = lax.rsqrt(deg)[:, None]
    dinv_ref[...] = dinv
    t0_ref[...] = dinv * xw0


def _layer_norm_relu(pre, g, beta):
    mu = jnp.mean(pre, axis=-1, keepdims=True)
    var = jnp.mean(jnp.square(pre - mu), axis=-1, keepdims=True)
    o = (pre - mu) * lax.rsqrt(var + 1e-5) * g + beta
    return jnp.maximum(o, 0.0)


def _make_cell_body(has_hsum_in, want_hsum_out):
    def body(*refs):
        (agg_ref, xw_ref, dinv_ref, b_ref, g_ref, beta_ref) = refs[:6]
        pos = 6
        if has_hsum_in:
            hin_ref = refs[pos]
            pos += 1
        wn_ref = refs[pos]
        pos += 1
        t_ref, xwn_ref = refs[pos], refs[pos + 1]
        pos += 2
        if want_hsum_out:
            hout_ref = refs[pos]

        dinv = dinv_ref[...]
        agg = agg_ref[...]
        a = agg[0] + agg[1]
        pre = dinv * a + (dinv * dinv) * xw_ref[...] + b_ref[...]
        o = _layer_norm_relu(pre, g_ref[...], beta_ref[...])
        if has_hsum_in:
            hsum = hin_ref[...] + o
        else:
            hsum = o
        if want_hsum_out:
            hout_ref[...] = hsum
        xwn = jnp.dot(hsum, wn_ref[...], precision=_HI)
        xwn_ref[...] = xwn
        t_ref[...] = dinv * xwn

    return body


def _final_body(agg_ref, xw_ref, dinv_ref, b_ref, g_ref, beta_ref,
                batch_ref, pw_ref, pb_ref, out_ref, pool_acc):
    i = pl.program_id(0)
    dinv = dinv_ref[...]
    agg = agg_ref[...]
    a = agg[0] + agg[1]
    pre = dinv * a + (dinv * dinv) * xw_ref[...] + b_ref[...]
    o = _layer_norm_relu(pre, g_ref[...], beta_ref[...])
    gid = lax.broadcasted_iota(jnp.int32, (1, NUM_GRAPHS), 1)
    onehot = (batch_ref[...] == gid).astype(jnp.float32)
    pooled = lax.dot_general(onehot, o, (((0,), (0,)), ((), ())),
                             precision=_HI)

    @pl.when(i == 0)
    def _():
        pool_acc[...] = jnp.zeros_like(pool_acc)

    pool_acc[...] += pooled

    @pl.when(i == pl.num_programs(0) - 1)
    def _():
        out_ref[...] = (jnp.dot(pool_acc[...], pw_ref[...], precision=_HI)
                        + pb_ref[...])


def _tc(body, grid, in_specs, out_specs, out_shape, *args, scratch_shapes=()):
    return pl.pallas_call(
        body, grid=grid, in_specs=in_specs, out_specs=out_specs,
        out_shape=out_shape, scratch_shapes=scratch_shapes,
        compiler_params=pltpu.CompilerParams(
            vmem_limit_bytes=60 * 1024 * 1024),
    )(*args)


def kernel(x, edge_index, batch, params):
    N, D = x.shape
    E = edge_index.shape[1]
    ch = 125
    src = edge_index[0].reshape(E // ch, ch)
    dst_flat = edge_index[1]
    dst = dst_flat.reshape(E // ch, ch)
    zeros_nd = jnp.zeros((N, D), jnp.float32)
    batch2 = batch.reshape(N, 1)
    cells = params["cells"]
    ncells = len(cells)

    deg_kernel = _make_deg_kernel(E, N)
    edge_kernel = _make_edge_kernel(E, N, D)

    deg_p = deg_kernel(dst_flat).reshape(NW, N)
    grid = (N // BS,)
    row = _row_spec
    dinv_spec = pl.BlockSpec((BS, 1), lambda i: (i, 0))
    w_spec = _full_spec((D, D))
    v_spec = _full_spec((D,))
    agg_spec = pl.BlockSpec((NC, BS, D), lambda i: (0, i, 0))
    nd_sds = jax.ShapeDtypeStruct((N, D), jnp.float32)

    xw, dinv, t = pl.pallas_call(
        _pre_body,
        out_shape=(nd_sds, jax.ShapeDtypeStruct((N, 1), jnp.float32),
                   nd_sds),
        compiler_params=pltpu.CompilerParams(
            vmem_limit_bytes=60 * 1024 * 1024),
    )(x, params["pre_W"], params["pre_b"], cells[0]["W"], deg_p)

    hsum = None
    for i in range(ncells):
        agg = edge_kernel(t, src, dst, zeros_nd)
        c = cells[i]
        if i < ncells - 1:
            has_hin = i > 0
            want_hout = i < ncells - 2
            ins = [agg, xw, dinv, c["b"], c["g"], c["beta"]]
            specs = [agg_spec, row(), dinv_spec, v_spec, v_spec, v_spec]
            if has_hin:
                ins.append(hsum)
                specs.append(row())
            ins.append(cells[i + 1]["W"])
            specs.append(w_spec)
            outs = [nd_sds, nd_sds]
            out_specs = [row(), row()]
            if want_hout:
                outs.append(nd_sds)
                out_specs.append(row())
            res = _tc(_make_cell_body(has_hin, want_hout), grid,
                      specs, tuple(out_specs), tuple(outs), *ins)
            if want_hout:
                t, xw, hsum = res
            else:
                t, xw = res
        else:
            n_out = params["post_W"].shape[1]
            out = _tc(
                _final_body, grid,
                [agg_spec, row(), dinv_spec, v_spec, v_spec, v_spec,
                 pl.BlockSpec((BS, 1), lambda i: (i, 0)),
                 _full_spec((D, n_out)), _full_spec((n_out,))],
                pl.BlockSpec((NUM_GRAPHS, n_out), lambda i: (0, 0)),
                jax.ShapeDtypeStruct((NUM_GRAPHS, n_out), jnp.float32),
                agg, xw, dinv, c["b"], c["g"], c["beta"], batch2,
                params["post_W"], params["post_b"],
                scratch_shapes=[pltpu.VMEM((NUM_GRAPHS, D), jnp.float32)])
    return out
